# baseline (speedup 1.0000x reference)
import functools

import jax
import jax.numpy as jnp
from jax import lax
from jax.experimental import pallas as pl
from jax.experimental.pallas import tpu as pltpu
from jax.experimental.pallas import tpu_sc as plsc

B, N, DIN, DOUT, PHID, K = 8, 2048, 128, 128, 64, 16
D2 = 2 * DOUT
RA = 256
RC = 256
CH = 256


def _oddeven_merge_sort_pairs(n):
    pairs = []

    def merge(lo, length, r):
        step = r * 2
        if step < length:
            merge(lo, length, step)
            merge(lo + r, length, step)
            for i in range(lo + r, lo + length - r, step):
                pairs.append((i, i + r))
        else:
            pairs.append((lo, lo + r))

    def sort(lo, length):
        if length > 1:
            mid = length // 2
            sort(lo, mid)
            sort(lo + mid, mid)
            merge(lo, length, 1)

    sort(0, n)
    needed = set(range(8))
    keep = []
    for i, j in reversed(pairs):
        if i in needed or j in needed:
            keep.append((i, j))
            needed |= {i, j}
    keep.reverse()
    return keep


_OE_PAIRS = _oddeven_merge_sort_pairs(K)

def _prep_body(x_ref, posp_ref, post_ref, wqkv_ref, w1_ref, b1_ref,
               w2_ref, b2_ref, a1w_ref, a1b_ref, a2w_ref, a2b_ref,
               tab_ref, idx_ref):
    b = pl.program_id(0)
    x = x_ref[0]
    posp = posp_ref[0]
    post = post_ref[0]

    qkv = jnp.dot(x, wqkv_ref[...])
    q = qkv[:, :DOUT]
    k = qkv[:, DOUT:2 * DOUT]
    v = qkv[:, 2 * DOUT:]
    h = jnp.maximum(jnp.dot(posp, w1_ref[...]) + b1_ref[...], 0.0)
    rpe = jnp.dot(h, w2_ref[...]) + b2_ref[...]
    st = q - k + rpe
    hh = jnp.maximum(
        jnp.dot(st, a1w_ref[...], preferred_element_type=jnp.float32)
        + a1b_ref[...], 0.0)
    sim = jnp.dot(hh, a2w_ref[...],
                  preferred_element_type=jnp.float32) + a2b_ref[...]
    ee = jnp.exp(sim)
    zz = ee * (v + rpe)
    s16 = lax.bitcast_convert_type(ee.astype(jnp.bfloat16), jnp.uint16)
    w16 = lax.bitcast_convert_type(zz.astype(jnp.bfloat16), jnp.uint16)
    packed = (w16.astype(jnp.uint32) << 16) | s16.astype(jnp.uint32)
    tab_ref[0] = lax.bitcast_convert_type(packed, jnp.int32)

    inner = jnp.dot(posp, post)
    quad_full = (post[0:1] * post[0:1] + post[1:2] * post[1:2]
                 + post[2:3] * post[2:3])
    quad_blk = (posp[:, 0:1] * posp[:, 0:1] + posp[:, 1:2] * posp[:, 1:2]
                + posp[:, 2:3] * posp[:, 2:3])
    dist = inner * (-2.0) + quad_full + quad_blk

    layers = []
    for l in range(K):
        d_l = lax.bitcast_convert_type(dist[:, l * 128:(l + 1) * 128],
                                       jnp.int32) + 0x08000000
        layers.append(lax.bitcast_convert_type((d_l & ~15) | l, jnp.float32))
    for i, j in _OE_PAIRS:
        a, c = layers[i], layers[j]
        layers[i] = jnp.minimum(a, c)
        layers[j] = jnp.maximum(a, c)
    heads = layers[:8]
    lane_f = lax.broadcasted_iota(jnp.int32, (RA, 128), 1).astype(jnp.float32)
    off = b * N
    bigk = lax.bitcast_convert_type(jnp.int32(0x7F000000), jnp.float32)
    cols = []
    for _ in range(K):
        m = jnp.min(heads[0], axis=1, keepdims=True)
        eqc = heads[0] == m
        col = jnp.min(jnp.where(eqc, lane_f, 4096.0), axis=1)[:, None]
        layer = lax.bitcast_convert_type(m, jnp.int32) & 15
        cols.append(layer * 128 + col.astype(jnp.int32) + off)
        for l in range(7):
            heads[l] = jnp.where(eqc, heads[l + 1], heads[l])
        heads[7] = jnp.where(eqc, bigk, heads[7])
    idx_ref[0] = jnp.concatenate(cols, axis=1)


_prep_call = pl.pallas_call(
    _prep_body,
    grid=(B, N // RA),
    in_specs=[
        pl.BlockSpec((1, RA, DIN), lambda b, i: (b, i, 0)),
        pl.BlockSpec((1, RA, 128), lambda b, i: (b, i, 0)),
        pl.BlockSpec((1, 128, N), lambda b, i: (b, 0, 0)),
        pl.BlockSpec((DIN, 3 * DOUT), lambda b, i: (0, 0)),
        pl.BlockSpec((128, PHID), lambda b, i: (0, 0)),
        pl.BlockSpec((1, PHID), lambda b, i: (0, 0)),
        pl.BlockSpec((PHID, DOUT), lambda b, i: (0, 0)),
        pl.BlockSpec((1, DOUT), lambda b, i: (0, 0)),
        pl.BlockSpec((DOUT, 4 * DOUT), lambda b, i: (0, 0)),
        pl.BlockSpec((1, 4 * DOUT), lambda b, i: (0, 0)),
        pl.BlockSpec((4 * DOUT, DOUT), lambda b, i: (0, 0)),
        pl.BlockSpec((1, DOUT), lambda b, i: (0, 0)),
    ],
    out_specs=[
        pl.BlockSpec((1, RA, DOUT), lambda b, i: (b, i, 0)),
        pl.BlockSpec((1, RA, K), lambda b, i: (b, i, 0)),
    ],
    out_shape=[
        jax.ShapeDtypeStruct((B, N, DOUT), jnp.int32),
        jax.ShapeDtypeStruct((B, N, K), jnp.int32),
    ],
)


@functools.lru_cache(maxsize=None)
def _make_gather(nrows):
    sc = plsc.get_sparse_core_info()
    nw = sc.num_cores * sc.num_subcores
    mesh = plsc.VectorSubcoreMesh(core_axis_name="c", subcore_axis_name="s")
    per_w = nrows // nw
    nch = per_w // CH

    @functools.partial(
        pl.kernel,
        mesh=mesh,
        out_type=jax.ShapeDtypeStruct((nrows, DOUT), jnp.int32),
        scratch_types=[
            pltpu.VMEM((per_w,), jnp.int32),
            pltpu.VMEM((2, CH, DOUT), jnp.int32),
            pltpu.SemaphoreType.DMA,
            pltpu.SemaphoreType.DMA,
            pltpu.SemaphoreType.DMA,
        ],
    )
    def gather_k(table_hbm, idx_hbm, out_hbm, idx_v, rows_v, gsem, os0, os1):
        wid = lax.axis_index("s") * sc.num_cores + lax.axis_index("c")
        base = wid * per_w
        pltpu.sync_copy(idx_hbm.at[pl.ds(base, per_w)], idx_v)
        osem = (os0, os1)
        wb = [None, None]
        for g in range(nch):
            c = g & 1
            if wb[c] is not None:
                wb[c].wait()
            pltpu.async_copy(table_hbm.at[idx_v.at[pl.ds(g * CH, CH)]],
                             rows_v.at[c], gsem).wait()
            wb[c] = pltpu.async_copy(
                rows_v.at[c], out_hbm.at[pl.ds(base + g * CH, CH)], osem[c])
        for c in (0, 1):
            if wb[c] is not None:
                wb[c].wait()

    return gather_k


def _agg_body(g_ref, out_ref):
    u = lax.bitcast_convert_type(g_ref[0], jnp.uint32)
    e = lax.bitcast_convert_type((u & 0xFFFF).astype(jnp.uint16),
                                 jnp.bfloat16).astype(jnp.float32)
    z = lax.bitcast_convert_type((u >> 16).astype(jnp.uint16),
                                 jnp.bfloat16).astype(jnp.float32)
    se = jnp.sum(e.reshape(RC, K, DOUT), axis=1)
    sz = jnp.sum(z.reshape(RC, K, DOUT), axis=1)
    out_ref[0] = sz / se


_agg_call = pl.pallas_call(
    _agg_body,
    grid=(B, N // RC),
    in_specs=[
        pl.BlockSpec((1, RC * K, DOUT), lambda b, i: (b, i, 0)),
    ],
    out_specs=pl.BlockSpec((1, RC, DOUT), lambda b, i: (b, i, 0)),
    out_shape=jax.ShapeDtypeStruct((B, N, DOUT), jnp.float32),
)


def kernel(x, pos, W_qkv, W1, b1, W2, b2, A1, a1, A2, a2):
    posp = jnp.pad(pos, ((0, 0), (0, 0), (0, 128 - 3)))
    post = jnp.swapaxes(posp, 1, 2)
    w1p = jnp.pad(W1, ((0, 128 - 3), (0, 0)))
    tab, idx = _prep_call(x, posp, post, W_qkv, w1p,
                          b1.reshape(1, -1), W2, b2.reshape(1, -1),
                          A1, a1.reshape(1, -1), A2, a2.reshape(1, -1))
    g = _make_gather(B * N * K)(tab.reshape(B * N, DOUT), idx.reshape(-1))
    out = _agg_call(g.reshape(B, N * K, DOUT))
    return out


# speedup vs baseline: 1.1294x; 1.1294x over previous
import functools

import jax
import jax.numpy as jnp
from jax import lax
from jax.experimental import pallas as pl
from jax.experimental.pallas import tpu as pltpu
from jax.experimental.pallas import tpu_sc as plsc

B, N, DIN, DOUT, PHID, K = 8, 2048, 128, 128, 64, 16
D2 = 2 * DOUT
RA = 256
RC = 256
CH = 256


def _oddeven_merge_sort_pairs(n):
    pairs = []

    def merge(lo, length, r):
        step = r * 2
        if step < length:
            merge(lo, length, step)
            merge(lo + r, length, step)
            for i in range(lo + r, lo + length - r, step):
                pairs.append((i, i + r))
        else:
            pairs.append((lo, lo + r))

    def sort(lo, length):
        if length > 1:
            mid = length // 2
            sort(lo, mid)
            sort(lo + mid, mid)
            merge(lo, length, 1)

    sort(0, n)
    needed = set(range(8))
    keep = []
    for i, j in reversed(pairs):
        if i in needed or j in needed:
            keep.append((i, j))
            needed |= {i, j}
    keep.reverse()
    return keep


_OE_PAIRS = _oddeven_merge_sort_pairs(K)

NB = 2
BG = B // NB


def _prep_body(x_ref, posp_ref, post_ref, wqkv_ref, w1_ref, b1_ref,
               w2_ref, b2_ref, a1w_ref, a1b_ref, a2w_ref, a2b_ref,
               tab_ref, idx_ref):
    b = pl.program_id(0)
    x = x_ref[0]
    posp = posp_ref[0]
    post = post_ref[0]

    qkv = jnp.dot(x, wqkv_ref[...])
    q = qkv[:, :DOUT]
    k = qkv[:, DOUT:2 * DOUT]
    v = qkv[:, 2 * DOUT:]
    h = jnp.maximum(jnp.dot(posp, w1_ref[...]) + b1_ref[...], 0.0)
    rpe = jnp.dot(h, w2_ref[...]) + b2_ref[...]
    st = q - k + rpe
    hh = jnp.maximum(
        jnp.dot(st, a1w_ref[...], preferred_element_type=jnp.float32)
        + a1b_ref[...], 0.0)
    sim = jnp.dot(hh, a2w_ref[...],
                  preferred_element_type=jnp.float32) + a2b_ref[...]
    ee = jnp.exp(sim)
    zz = ee * (v + rpe)
    s16 = lax.bitcast_convert_type(ee.astype(jnp.bfloat16), jnp.uint16)
    w16 = lax.bitcast_convert_type(zz.astype(jnp.bfloat16), jnp.uint16)
    packed = (w16.astype(jnp.uint32) << 16) | s16.astype(jnp.uint32)
    tab_ref[0] = lax.bitcast_convert_type(packed, jnp.int32)

    inner = jnp.dot(posp, post)
    quad_full = (post[0:1] * post[0:1] + post[1:2] * post[1:2]
                 + post[2:3] * post[2:3])
    quad_blk = (posp[:, 0:1] * posp[:, 0:1] + posp[:, 1:2] * posp[:, 1:2]
                + posp[:, 2:3] * posp[:, 2:3])
    dist = inner * (-2.0) + quad_full + quad_blk

    layers = []
    for l in range(K):
        d_l = lax.bitcast_convert_type(dist[:, l * 128:(l + 1) * 128],
                                       jnp.int32) + 0x08000000
        layers.append(lax.bitcast_convert_type((d_l & ~15) | l, jnp.float32))
    for i, j in _OE_PAIRS:
        a, c = layers[i], layers[j]
        layers[i] = jnp.minimum(a, c)
        layers[j] = jnp.maximum(a, c)
    heads = layers[:8]
    lane_f = lax.broadcasted_iota(jnp.int32, (RA, 128), 1).astype(jnp.float32)
    off = b * N
    bigk = lax.bitcast_convert_type(jnp.int32(0x7F000000), jnp.float32)
    cols = []
    for _ in range(K):
        m = jnp.min(heads[0], axis=1, keepdims=True)
        eqc = heads[0] == m
        col = jnp.min(jnp.where(eqc, lane_f, 4096.0), axis=1)[:, None]
        layer = lax.bitcast_convert_type(m, jnp.int32) & 15
        cols.append(layer * 128 + col.astype(jnp.int32) + off)
        for l in range(7):
            heads[l] = jnp.where(eqc, heads[l + 1], heads[l])
        heads[7] = jnp.where(eqc, bigk, heads[7])
    idx_ref[0] = jnp.concatenate(cols, axis=1)


_prep_call = pl.pallas_call(
    _prep_body,
    grid=(BG, N // RA),
    in_specs=[
        pl.BlockSpec((1, RA, DIN), lambda b, i: (b, i, 0)),
        pl.BlockSpec((1, RA, 128), lambda b, i: (b, i, 0)),
        pl.BlockSpec((1, 128, N), lambda b, i: (b, 0, 0)),
        pl.BlockSpec((DIN, 3 * DOUT), lambda b, i: (0, 0)),
        pl.BlockSpec((128, PHID), lambda b, i: (0, 0)),
        pl.BlockSpec((1, PHID), lambda b, i: (0, 0)),
        pl.BlockSpec((PHID, DOUT), lambda b, i: (0, 0)),
        pl.BlockSpec((1, DOUT), lambda b, i: (0, 0)),
        pl.BlockSpec((DOUT, 4 * DOUT), lambda b, i: (0, 0)),
        pl.BlockSpec((1, 4 * DOUT), lambda b, i: (0, 0)),
        pl.BlockSpec((4 * DOUT, DOUT), lambda b, i: (0, 0)),
        pl.BlockSpec((1, DOUT), lambda b, i: (0, 0)),
    ],
    out_specs=[
        pl.BlockSpec((1, RA, DOUT), lambda b, i: (b, i, 0)),
        pl.BlockSpec((1, RA, K), lambda b, i: (b, i, 0)),
    ],
    out_shape=[
        jax.ShapeDtypeStruct((BG, N, DOUT), jnp.int32),
        jax.ShapeDtypeStruct((BG, N, K), jnp.int32),
    ],
)


@functools.lru_cache(maxsize=None)
def _make_gather(nrows):
    sc = plsc.get_sparse_core_info()
    nw = sc.num_cores * sc.num_subcores
    mesh = plsc.VectorSubcoreMesh(core_axis_name="c", subcore_axis_name="s")
    per_w = nrows // nw
    nch = per_w // CH

    @functools.partial(
        pl.kernel,
        mesh=mesh,
        out_type=jax.ShapeDtypeStruct((nrows, DOUT), jnp.int32),
        scratch_types=[
            pltpu.VMEM((per_w,), jnp.int32),
            pltpu.VMEM((2, CH, DOUT), jnp.int32),
            pltpu.SemaphoreType.DMA,
            pltpu.SemaphoreType.DMA,
            pltpu.SemaphoreType.DMA,
        ],
    )
    def gather_k(table_hbm, idx_hbm, out_hbm, idx_v, rows_v, gsem, os0, os1):
        wid = lax.axis_index("s") * sc.num_cores + lax.axis_index("c")
        base = wid * per_w
        pltpu.sync_copy(idx_hbm.at[pl.ds(base, per_w)], idx_v)
        osem = (os0, os1)
        wb = [None, None]
        for g in range(nch):
            c = g & 1
            if wb[c] is not None:
                wb[c].wait()
            pltpu.async_copy(table_hbm.at[idx_v.at[pl.ds(g * CH, CH)]],
                             rows_v.at[c], gsem).wait()
            wb[c] = pltpu.async_copy(
                rows_v.at[c], out_hbm.at[pl.ds(base + g * CH, CH)], osem[c])
        for c in (0, 1):
            if wb[c] is not None:
                wb[c].wait()

    return gather_k


def _agg_body(g_ref, out_ref):
    u = lax.bitcast_convert_type(g_ref[0], jnp.uint32)
    e = lax.bitcast_convert_type((u & 0xFFFF).astype(jnp.uint16),
                                 jnp.bfloat16).astype(jnp.float32)
    z = lax.bitcast_convert_type((u >> 16).astype(jnp.uint16),
                                 jnp.bfloat16).astype(jnp.float32)
    se = jnp.sum(e.reshape(RC, K, DOUT), axis=1)
    sz = jnp.sum(z.reshape(RC, K, DOUT), axis=1)
    out_ref[0] = sz / se


_agg_call = pl.pallas_call(
    _agg_body,
    grid=(BG, N // RC),
    in_specs=[
        pl.BlockSpec((1, RC * K, DOUT), lambda b, i: (b, i, 0)),
    ],
    out_specs=pl.BlockSpec((1, RC, DOUT), lambda b, i: (b, i, 0)),
    out_shape=jax.ShapeDtypeStruct((BG, N, DOUT), jnp.float32),
)


def kernel(x, pos, W_qkv, W1, b1, W2, b2, A1, a1, A2, a2):
    posp = jnp.pad(pos, ((0, 0), (0, 0), (0, 128 - 3)))
    post = jnp.swapaxes(posp, 1, 2)
    w1p = jnp.pad(W1, ((0, 128 - 3), (0, 0)))
    gather = _make_gather(BG * N * K)
    tabs = []
    for g in range(NB):
        s = slice(g * BG, (g + 1) * BG)
        tabs.append(_prep_call(x[s], posp[s], post[s], W_qkv, w1p,
                               b1.reshape(1, -1), W2, b2.reshape(1, -1),
                               A1, a1.reshape(1, -1), A2, a2.reshape(1, -1)))
    outs = []
    for tab, idx in tabs:
        rows = gather(tab.reshape(BG * N, DOUT), idx.reshape(-1))
        outs.append(_agg_call(rows.reshape(BG, N * K, DOUT)))
    return jnp.concatenate(outs, axis=0)


# speedup vs baseline: 1.1337x; 1.0038x over previous
import functools

import jax
import jax.numpy as jnp
from jax import lax
from jax.experimental import pallas as pl
from jax.experimental.pallas import tpu as pltpu
from jax.experimental.pallas import tpu_sc as plsc

B, N, DIN, DOUT, PHID, K = 8, 2048, 128, 128, 64, 16
D2 = 2 * DOUT
RA = 256
RC = 256
CH = 256


def _oddeven_merge_sort_pairs(n):
    pairs = []

    def merge(lo, length, r):
        step = r * 2
        if step < length:
            merge(lo, length, step)
            merge(lo + r, length, step)
            for i in range(lo + r, lo + length - r, step):
                pairs.append((i, i + r))
        else:
            pairs.append((lo, lo + r))

    def sort(lo, length):
        if length > 1:
            mid = length // 2
            sort(lo, mid)
            sort(lo + mid, mid)
            merge(lo, length, 1)

    sort(0, n)
    needed = set(range(8))
    keep = []
    for i, j in reversed(pairs):
        if i in needed or j in needed:
            keep.append((i, j))
            needed |= {i, j}
    keep.reverse()
    return keep


_OE_PAIRS = _oddeven_merge_sort_pairs(K)

NB = 4
BG = B // NB


def _prep_body(x_ref, posp_ref, post_ref, wqkv_ref, w1_ref, b1_ref,
               w2_ref, b2_ref, a1w_ref, a1b_ref, a2w_ref, a2b_ref,
               tab_ref, idx_ref):
    b = pl.program_id(0)
    x = x_ref[0]
    posp = posp_ref[0]
    post = post_ref[0]

    qkv = jnp.dot(x, wqkv_ref[...])
    q = qkv[:, :DOUT]
    k = qkv[:, DOUT:2 * DOUT]
    v = qkv[:, 2 * DOUT:]
    h = jnp.maximum(jnp.dot(posp, w1_ref[...]) + b1_ref[...], 0.0)
    rpe = jnp.dot(h, w2_ref[...]) + b2_ref[...]
    st = q - k + rpe
    hh = jnp.maximum(
        jnp.dot(st, a1w_ref[...], preferred_element_type=jnp.float32)
        + a1b_ref[...], 0.0)
    sim = jnp.dot(hh, a2w_ref[...],
                  preferred_element_type=jnp.float32) + a2b_ref[...]
    ee = jnp.exp(sim)
    zz = ee * (v + rpe)
    s16 = lax.bitcast_convert_type(ee.astype(jnp.bfloat16), jnp.uint16)
    w16 = lax.bitcast_convert_type(zz.astype(jnp.bfloat16), jnp.uint16)
    packed = (w16.astype(jnp.uint32) << 16) | s16.astype(jnp.uint32)
    tab_ref[0] = lax.bitcast_convert_type(packed, jnp.int32)

    inner = jnp.dot(posp, post)
    quad_full = (post[0:1] * post[0:1] + post[1:2] * post[1:2]
                 + post[2:3] * post[2:3])
    quad_blk = (posp[:, 0:1] * posp[:, 0:1] + posp[:, 1:2] * posp[:, 1:2]
                + posp[:, 2:3] * posp[:, 2:3])
    dist = inner * (-2.0) + quad_full + quad_blk

    layers = []
    for l in range(K):
        d_l = lax.bitcast_convert_type(dist[:, l * 128:(l + 1) * 128],
                                       jnp.int32) + 0x08000000
        layers.append(lax.bitcast_convert_type((d_l & ~15) | l, jnp.float32))
    for i, j in _OE_PAIRS:
        a, c = layers[i], layers[j]
        layers[i] = jnp.minimum(a, c)
        layers[j] = jnp.maximum(a, c)
    heads = layers[:8]
    lane_f = lax.broadcasted_iota(jnp.int32, (RA, 128), 1).astype(jnp.float32)
    off = b * N
    bigk = lax.bitcast_convert_type(jnp.int32(0x7F000000), jnp.float32)
    cols = []
    for _ in range(K):
        m = jnp.min(heads[0], axis=1, keepdims=True)
        eqc = heads[0] == m
        col = jnp.min(jnp.where(eqc, lane_f, 4096.0), axis=1)[:, None]
        layer = lax.bitcast_convert_type(m, jnp.int32) & 15
        cols.append(layer * 128 + col.astype(jnp.int32) + off)
        for l in range(7):
            heads[l] = jnp.where(eqc, heads[l + 1], heads[l])
        heads[7] = jnp.where(eqc, bigk, heads[7])
    idx_ref[0] = jnp.concatenate(cols, axis=1)


_prep_call = pl.pallas_call(
    _prep_body,
    grid=(BG, N // RA),
    in_specs=[
        pl.BlockSpec((1, RA, DIN), lambda b, i: (b, i, 0)),
        pl.BlockSpec((1, RA, 128), lambda b, i: (b, i, 0)),
        pl.BlockSpec((1, 128, N), lambda b, i: (b, 0, 0)),
        pl.BlockSpec((DIN, 3 * DOUT), lambda b, i: (0, 0)),
        pl.BlockSpec((128, PHID), lambda b, i: (0, 0)),
        pl.BlockSpec((1, PHID), lambda b, i: (0, 0)),
        pl.BlockSpec((PHID, DOUT), lambda b, i: (0, 0)),
        pl.BlockSpec((1, DOUT), lambda b, i: (0, 0)),
        pl.BlockSpec((DOUT, 4 * DOUT), lambda b, i: (0, 0)),
        pl.BlockSpec((1, 4 * DOUT), lambda b, i: (0, 0)),
        pl.BlockSpec((4 * DOUT, DOUT), lambda b, i: (0, 0)),
        pl.BlockSpec((1, DOUT), lambda b, i: (0, 0)),
    ],
    out_specs=[
        pl.BlockSpec((1, RA, DOUT), lambda b, i: (b, i, 0)),
        pl.BlockSpec((1, RA, K), lambda b, i: (b, i, 0)),
    ],
    out_shape=[
        jax.ShapeDtypeStruct((BG, N, DOUT), jnp.int32),
        jax.ShapeDtypeStruct((BG, N, K), jnp.int32),
    ],
)


@functools.lru_cache(maxsize=None)
def _make_gather(nrows):
    sc = plsc.get_sparse_core_info()
    nw = sc.num_cores * sc.num_subcores
    mesh = plsc.VectorSubcoreMesh(core_axis_name="c", subcore_axis_name="s")
    per_w = nrows // nw
    nch = per_w // CH

    @functools.partial(
        pl.kernel,
        mesh=mesh,
        out_type=jax.ShapeDtypeStruct((nrows, DOUT), jnp.int32),
        scratch_types=[
            pltpu.VMEM((per_w,), jnp.int32),
            pltpu.VMEM((2, CH, DOUT), jnp.int32),
            pltpu.SemaphoreType.DMA,
            pltpu.SemaphoreType.DMA,
            pltpu.SemaphoreType.DMA,
        ],
    )
    def gather_k(table_hbm, idx_hbm, out_hbm, idx_v, rows_v, gsem, os0, os1):
        wid = lax.axis_index("s") * sc.num_cores + lax.axis_index("c")
        base = wid * per_w
        pltpu.sync_copy(idx_hbm.at[pl.ds(base, per_w)], idx_v)
        osem = (os0, os1)
        wb = [None, None]
        for g in range(nch):
            c = g & 1
            if wb[c] is not None:
                wb[c].wait()
            pltpu.async_copy(table_hbm.at[idx_v.at[pl.ds(g * CH, CH)]],
                             rows_v.at[c], gsem).wait()
            wb[c] = pltpu.async_copy(
                rows_v.at[c], out_hbm.at[pl.ds(base + g * CH, CH)], osem[c])
        for c in (0, 1):
            if wb[c] is not None:
                wb[c].wait()

    return gather_k


def _agg_body(g_ref, out_ref):
    u = lax.bitcast_convert_type(g_ref[0], jnp.uint32)
    e = lax.bitcast_convert_type((u & 0xFFFF).astype(jnp.uint16),
                                 jnp.bfloat16).astype(jnp.float32)
    z = lax.bitcast_convert_type((u >> 16).astype(jnp.uint16),
                                 jnp.bfloat16).astype(jnp.float32)
    se = jnp.sum(e.reshape(RC, K, DOUT), axis=1)
    sz = jnp.sum(z.reshape(RC, K, DOUT), axis=1)
    out_ref[0] = sz / se


_agg_call = pl.pallas_call(
    _agg_body,
    grid=(BG, N // RC),
    in_specs=[
        pl.BlockSpec((1, RC * K, DOUT), lambda b, i: (b, i, 0)),
    ],
    out_specs=pl.BlockSpec((1, RC, DOUT), lambda b, i: (b, i, 0)),
    out_shape=jax.ShapeDtypeStruct((BG, N, DOUT), jnp.float32),
)


def kernel(x, pos, W_qkv, W1, b1, W2, b2, A1, a1, A2, a2):
    posp = jnp.pad(pos, ((0, 0), (0, 0), (0, 128 - 3)))
    post = jnp.swapaxes(posp, 1, 2)
    w1p = jnp.pad(W1, ((0, 128 - 3), (0, 0)))
    gather = _make_gather(BG * N * K)
    tabs = []
    for g in range(NB):
        s = slice(g * BG, (g + 1) * BG)
        tabs.append(_prep_call(x[s], posp[s], post[s], W_qkv, w1p,
                               b1.reshape(1, -1), W2, b2.reshape(1, -1),
                               A1, a1.reshape(1, -1), A2, a2.reshape(1, -1)))
    outs = []
    for tab, idx in tabs:
        rows = gather(tab.reshape(BG * N, DOUT), idx.reshape(-1))
        outs.append(_agg_call(rows.reshape(BG, N * K, DOUT)))
    return jnp.concatenate(outs, axis=0)


# speedup vs baseline: 1.1442x; 1.0092x over previous
import functools

import jax
import jax.numpy as jnp
from jax import lax
from jax.experimental import pallas as pl
from jax.experimental.pallas import tpu as pltpu
from jax.experimental.pallas import tpu_sc as plsc

B, N, DIN, DOUT, PHID, K = 8, 2048, 128, 128, 64, 16
D2 = 2 * DOUT
RA = 256
RC = 256
CH = 256


def _oddeven_merge_sort_pairs(n):
    pairs = []

    def merge(lo, length, r):
        step = r * 2
        if step < length:
            merge(lo, length, step)
            merge(lo + r, length, step)
            for i in range(lo + r, lo + length - r, step):
                pairs.append((i, i + r))
        else:
            pairs.append((lo, lo + r))

    def sort(lo, length):
        if length > 1:
            mid = length // 2
            sort(lo, mid)
            sort(lo + mid, mid)
            merge(lo, length, 1)

    sort(0, n)
    needed = set(range(8))
    keep = []
    for i, j in reversed(pairs):
        if i in needed or j in needed:
            keep.append((i, j))
            needed |= {i, j}
    keep.reverse()
    return keep


_OE_PAIRS = _oddeven_merge_sort_pairs(K)

NB = 4
BG = B // NB


def _prep_body(x_ref, posp_ref, post_ref, wqkv_ref, w1_ref, b1_ref,
               w2_ref, b2_ref, a1w_ref, a1b_ref, a2w_ref, a2b_ref,
               tab_ref, idx_ref):
    b = pl.program_id(0)
    x = x_ref[0]
    posp = posp_ref[0]
    post = post_ref[0]

    qkv = jnp.dot(x, wqkv_ref[...])
    q = qkv[:, :DOUT]
    k = qkv[:, DOUT:2 * DOUT]
    v = qkv[:, 2 * DOUT:]
    h = jnp.maximum(jnp.dot(posp, w1_ref[...]) + b1_ref[...], 0.0)
    rpe = jnp.dot(h, w2_ref[...]) + b2_ref[...]
    st = q - k + rpe
    hh = jnp.maximum(
        jnp.dot(st, a1w_ref[...], preferred_element_type=jnp.float32)
        + a1b_ref[...], 0.0)
    sim = jnp.dot(hh, a2w_ref[...],
                  preferred_element_type=jnp.float32) + a2b_ref[...]
    ee = jnp.exp(sim)
    zz = ee * (v + rpe)
    s16 = lax.bitcast_convert_type(ee.astype(jnp.bfloat16), jnp.uint16)
    w16 = lax.bitcast_convert_type(zz.astype(jnp.bfloat16), jnp.uint16)
    packed = (w16.astype(jnp.uint32) << 16) | s16.astype(jnp.uint32)
    tab_ref[0] = lax.bitcast_convert_type(packed, jnp.int32)

    inner = jnp.dot(posp, post)
    quad_full = (post[0:1] * post[0:1] + post[1:2] * post[1:2]
                 + post[2:3] * post[2:3])
    quad_blk = (posp[:, 0:1] * posp[:, 0:1] + posp[:, 1:2] * posp[:, 1:2]
                + posp[:, 2:3] * posp[:, 2:3])
    dist = inner * (-2.0) + quad_full + quad_blk

    RG = 32
    lane_f = lax.broadcasted_iota(jnp.int32, (RG, 128), 1).astype(jnp.float32)
    off = b * N
    bigk = lax.bitcast_convert_type(jnp.int32(0x7F000000), jnp.float32)
    idx_rows = []
    for rg in range(RA // RG):
        d = dist[rg * RG:(rg + 1) * RG]
        layers = []
        for l in range(K):
            d_l = lax.bitcast_convert_type(d[:, l * 128:(l + 1) * 128],
                                           jnp.int32) + 0x08000000
            layers.append(
                lax.bitcast_convert_type((d_l & ~15) | l, jnp.float32))
        for i, j in _OE_PAIRS:
            a, c = layers[i], layers[j]
            layers[i] = jnp.minimum(a, c)
            layers[j] = jnp.maximum(a, c)
        heads = layers[:8]
        cols = []
        for _ in range(K):
            m = jnp.min(heads[0], axis=1, keepdims=True)
            eqc = heads[0] == m
            col = jnp.min(jnp.where(eqc, lane_f, 4096.0), axis=1)[:, None]
            layer = lax.bitcast_convert_type(m, jnp.int32) & 15
            cols.append(layer * 128 + col.astype(jnp.int32) + off)
            for l in range(7):
                heads[l] = jnp.where(eqc, heads[l + 1], heads[l])
            heads[7] = jnp.where(eqc, bigk, heads[7])
        idx_rows.append(jnp.concatenate(cols, axis=1))
    idx_ref[0] = jnp.concatenate(idx_rows, axis=0)


_prep_call = pl.pallas_call(
    _prep_body,
    grid=(BG, N // RA),
    in_specs=[
        pl.BlockSpec((1, RA, DIN), lambda b, i: (b, i, 0)),
        pl.BlockSpec((1, RA, 128), lambda b, i: (b, i, 0)),
        pl.BlockSpec((1, 128, N), lambda b, i: (b, 0, 0)),
        pl.BlockSpec((DIN, 3 * DOUT), lambda b, i: (0, 0)),
        pl.BlockSpec((128, PHID), lambda b, i: (0, 0)),
        pl.BlockSpec((1, PHID), lambda b, i: (0, 0)),
        pl.BlockSpec((PHID, DOUT), lambda b, i: (0, 0)),
        pl.BlockSpec((1, DOUT), lambda b, i: (0, 0)),
        pl.BlockSpec((DOUT, 4 * DOUT), lambda b, i: (0, 0)),
        pl.BlockSpec((1, 4 * DOUT), lambda b, i: (0, 0)),
        pl.BlockSpec((4 * DOUT, DOUT), lambda b, i: (0, 0)),
        pl.BlockSpec((1, DOUT), lambda b, i: (0, 0)),
    ],
    out_specs=[
        pl.BlockSpec((1, RA, DOUT), lambda b, i: (b, i, 0)),
        pl.BlockSpec((1, RA, K), lambda b, i: (b, i, 0)),
    ],
    out_shape=[
        jax.ShapeDtypeStruct((BG, N, DOUT), jnp.int32),
        jax.ShapeDtypeStruct((BG, N, K), jnp.int32),
    ],
)


@functools.lru_cache(maxsize=None)
def _make_gather(nrows):
    sc = plsc.get_sparse_core_info()
    nw = sc.num_cores * sc.num_subcores
    mesh = plsc.VectorSubcoreMesh(core_axis_name="c", subcore_axis_name="s")
    per_w = nrows // nw
    nch = per_w // CH

    @functools.partial(
        pl.kernel,
        mesh=mesh,
        out_type=jax.ShapeDtypeStruct((nrows, DOUT), jnp.int32),
        scratch_types=[
            pltpu.VMEM((per_w,), jnp.int32),
            pltpu.VMEM((2, CH, DOUT), jnp.int32),
            pltpu.SemaphoreType.DMA,
            pltpu.SemaphoreType.DMA,
            pltpu.SemaphoreType.DMA,
        ],
    )
    def gather_k(table_hbm, idx_hbm, out_hbm, idx_v, rows_v, gsem, os0, os1):
        wid = lax.axis_index("s") * sc.num_cores + lax.axis_index("c")
        base = wid * per_w
        pltpu.sync_copy(idx_hbm.at[pl.ds(base, per_w)], idx_v)
        osem = (os0, os1)
        wb = [None, None]
        for g in range(nch):
            c = g & 1
            if wb[c] is not None:
                wb[c].wait()
            pltpu.async_copy(table_hbm.at[idx_v.at[pl.ds(g * CH, CH)]],
                             rows_v.at[c], gsem).wait()
            wb[c] = pltpu.async_copy(
                rows_v.at[c], out_hbm.at[pl.ds(base + g * CH, CH)], osem[c])
        for c in (0, 1):
            if wb[c] is not None:
                wb[c].wait()

    return gather_k


def _agg_body(g_ref, out_ref):
    u = g_ref[0]
    e = lax.bitcast_convert_type(u << 16, jnp.float32)
    z = lax.bitcast_convert_type(
        lax.bitwise_and(u, jnp.int32(-65536)), jnp.float32)
    se = jnp.sum(e.reshape(RC, K, DOUT), axis=1)
    sz = jnp.sum(z.reshape(RC, K, DOUT), axis=1)
    out_ref[0] = sz / se


_agg_call = pl.pallas_call(
    _agg_body,
    grid=(BG, N // RC),
    in_specs=[
        pl.BlockSpec((1, RC * K, DOUT), lambda b, i: (b, i, 0)),
    ],
    out_specs=pl.BlockSpec((1, RC, DOUT), lambda b, i: (b, i, 0)),
    out_shape=jax.ShapeDtypeStruct((BG, N, DOUT), jnp.float32),
)


def kernel(x, pos, W_qkv, W1, b1, W2, b2, A1, a1, A2, a2):
    posp = jnp.pad(pos, ((0, 0), (0, 0), (0, 128 - 3)))
    post = jnp.swapaxes(posp, 1, 2)
    w1p = jnp.pad(W1, ((0, 128 - 3), (0, 0)))
    gather = _make_gather(BG * N * K)
    tabs = []
    for g in range(NB):
        s = slice(g * BG, (g + 1) * BG)
        tabs.append(_prep_call(x[s], posp[s], post[s], W_qkv, w1p,
                               b1.reshape(1, -1), W2, b2.reshape(1, -1),
                               A1, a1.reshape(1, -1), A2, a2.reshape(1, -1)))
    outs = []
    for tab, idx in tabs:
        rows = gather(tab.reshape(BG * N, DOUT), idx.reshape(-1))
        outs.append(_agg_call(rows.reshape(BG, N * K, DOUT)))
    return jnp.concatenate(outs, axis=0)


# speedup vs baseline: 1.1838x; 1.0346x over previous
import functools

import jax
import jax.numpy as jnp
from jax import lax
from jax.experimental import pallas as pl
from jax.experimental.pallas import tpu as pltpu
from jax.experimental.pallas import tpu_sc as plsc

B, N, DIN, DOUT, PHID, K = 8, 2048, 128, 128, 64, 16
D2 = 2 * DOUT
RA = 256
RC = 256
CH = 256


def _oddeven_merge_sort_pairs(n):
    pairs = []

    def merge(lo, length, r):
        step = r * 2
        if step < length:
            merge(lo, length, step)
            merge(lo + r, length, step)
            for i in range(lo + r, lo + length - r, step):
                pairs.append((i, i + r))
        else:
            pairs.append((lo, lo + r))

    def sort(lo, length):
        if length > 1:
            mid = length // 2
            sort(lo, mid)
            sort(lo + mid, mid)
            merge(lo, length, 1)

    sort(0, n)
    needed = set(range(8))
    keep = []
    for i, j in reversed(pairs):
        if i in needed or j in needed:
            keep.append((i, j))
            needed |= {i, j}
    keep.reverse()
    return keep


_OE_PAIRS = _oddeven_merge_sort_pairs(K)

NB = 4
BG = B // NB


def _prep_body(x_ref, posp_ref, post_ref, wqkv_ref, w1_ref, b1_ref,
               w2_ref, b2_ref, a1w_ref, a1b_ref, a2w_ref, a2b_ref,
               tab_ref, idx_ref):
    b = pl.program_id(0)
    x = x_ref[0]
    posp = posp_ref[0]
    post = post_ref[0]

    qkv = jnp.dot(x, wqkv_ref[...])
    q = qkv[:, :DOUT]
    k = qkv[:, DOUT:2 * DOUT]
    v = qkv[:, 2 * DOUT:]
    h = jnp.maximum(jnp.dot(posp, w1_ref[...]) + b1_ref[...], 0.0)
    rpe = jnp.dot(h, w2_ref[...]) + b2_ref[...]
    st = q - k + rpe
    hh = jnp.maximum(
        jnp.dot(st, a1w_ref[...], preferred_element_type=jnp.float32)
        + a1b_ref[...], 0.0)
    sim = jnp.dot(hh, a2w_ref[...],
                  preferred_element_type=jnp.float32) + a2b_ref[...]
    ee = jnp.exp(sim)
    zz = ee * (v + rpe)
    s16 = lax.bitcast_convert_type(ee.astype(jnp.bfloat16), jnp.uint16)
    w16 = lax.bitcast_convert_type(zz.astype(jnp.bfloat16), jnp.uint16)
    packed = (w16.astype(jnp.uint32) << 16) | s16.astype(jnp.uint32)
    tab_ref[0] = lax.bitcast_convert_type(packed, jnp.int32)

    inner = jnp.dot(posp, post)
    quad_full = (post[0:1] * post[0:1] + post[1:2] * post[1:2]
                 + post[2:3] * post[2:3])
    quad_blk = (posp[:, 0:1] * posp[:, 0:1] + posp[:, 1:2] * posp[:, 1:2]
                + posp[:, 2:3] * posp[:, 2:3])
    dist = inner * (-2.0) + quad_full + quad_blk

    RG = 32
    lane_f = lax.broadcasted_iota(jnp.int32, (RG, 128), 1).astype(jnp.float32)
    off = b * N
    bigk = lax.bitcast_convert_type(jnp.int32(0x7F000000), jnp.float32)
    idx_rows = []
    for rg in range(RA // RG):
        d = dist[rg * RG:(rg + 1) * RG]
        layers = []
        for l in range(K):
            d_l = lax.bitcast_convert_type(d[:, l * 128:(l + 1) * 128],
                                           jnp.int32) + 0x08000000
            layers.append(
                lax.bitcast_convert_type((d_l & ~15) | l, jnp.float32))
        for i, j in _OE_PAIRS:
            a, c = layers[i], layers[j]
            layers[i] = jnp.minimum(a, c)
            layers[j] = jnp.maximum(a, c)
        heads = layers[:8]
        cols = []
        for _ in range(K):
            m = jnp.min(heads[0], axis=1, keepdims=True)
            eqc = heads[0] == m
            col = jnp.min(jnp.where(eqc, lane_f, 4096.0), axis=1)[:, None]
            layer = lax.bitcast_convert_type(m, jnp.int32) & 15
            cols.append(layer * 128 + col.astype(jnp.int32) + off)
            for l in range(7):
                heads[l] = jnp.where(eqc, heads[l + 1], heads[l])
            heads[7] = jnp.where(eqc, bigk, heads[7])
        idx_rows.append(jnp.concatenate(cols, axis=1))
    idx_ref[0] = jnp.concatenate(idx_rows, axis=0)


_prep_call = pl.pallas_call(
    _prep_body,
    grid=(BG, N // RA),
    in_specs=[
        pl.BlockSpec((1, RA, DIN), lambda b, i: (b, i, 0)),
        pl.BlockSpec((1, RA, 8), lambda b, i: (b, i, 0)),
        pl.BlockSpec((1, 8, N), lambda b, i: (b, 0, 0)),
        pl.BlockSpec((DIN, 3 * DOUT), lambda b, i: (0, 0)),
        pl.BlockSpec((8, PHID), lambda b, i: (0, 0)),
        pl.BlockSpec((1, PHID), lambda b, i: (0, 0)),
        pl.BlockSpec((PHID, DOUT), lambda b, i: (0, 0)),
        pl.BlockSpec((1, DOUT), lambda b, i: (0, 0)),
        pl.BlockSpec((DOUT, 4 * DOUT), lambda b, i: (0, 0)),
        pl.BlockSpec((1, 4 * DOUT), lambda b, i: (0, 0)),
        pl.BlockSpec((4 * DOUT, DOUT), lambda b, i: (0, 0)),
        pl.BlockSpec((1, DOUT), lambda b, i: (0, 0)),
    ],
    out_specs=[
        pl.BlockSpec((1, RA, DOUT), lambda b, i: (b, i, 0)),
        pl.BlockSpec((1, RA, K), lambda b, i: (b, i, 0)),
    ],
    out_shape=[
        jax.ShapeDtypeStruct((BG, N, DOUT), jnp.int32),
        jax.ShapeDtypeStruct((BG, N, K), jnp.int32),
    ],
)


@functools.lru_cache(maxsize=None)
def _make_gather(nrows):
    sc = plsc.get_sparse_core_info()
    nw = sc.num_cores * sc.num_subcores
    mesh = plsc.VectorSubcoreMesh(core_axis_name="c", subcore_axis_name="s")
    per_w = nrows // nw
    nch = per_w // CH

    @functools.partial(
        pl.kernel,
        mesh=mesh,
        out_type=jax.ShapeDtypeStruct((nrows, DOUT), jnp.int32),
        scratch_types=[
            pltpu.VMEM((per_w,), jnp.int32),
            pltpu.VMEM((2, CH, DOUT), jnp.int32),
            pltpu.SemaphoreType.DMA,
            pltpu.SemaphoreType.DMA,
            pltpu.SemaphoreType.DMA,
        ],
    )
    def gather_k(table_hbm, idx_hbm, out_hbm, idx_v, rows_v, gsem, os0, os1):
        wid = lax.axis_index("s") * sc.num_cores + lax.axis_index("c")
        base = wid * per_w
        pltpu.sync_copy(idx_hbm.at[pl.ds(base, per_w)], idx_v)
        osem = (os0, os1)
        wb = [None, None]
        for g in range(nch):
            c = g & 1
            if wb[c] is not None:
                wb[c].wait()
            pltpu.async_copy(table_hbm.at[idx_v.at[pl.ds(g * CH, CH)]],
                             rows_v.at[c], gsem).wait()
            wb[c] = pltpu.async_copy(
                rows_v.at[c], out_hbm.at[pl.ds(base + g * CH, CH)], osem[c])
        for c in (0, 1):
            if wb[c] is not None:
                wb[c].wait()

    return gather_k


def _agg_body(g_ref, out_ref):
    u = g_ref[0]
    e = lax.bitcast_convert_type(u << 16, jnp.float32)
    z = lax.bitcast_convert_type(
        lax.bitwise_and(u, jnp.int32(-65536)), jnp.float32)
    se = jnp.sum(e.reshape(RC, K, DOUT), axis=1)
    sz = jnp.sum(z.reshape(RC, K, DOUT), axis=1)
    out_ref[0] = sz / se


_agg_call = pl.pallas_call(
    _agg_body,
    grid=(BG, N // RC),
    in_specs=[
        pl.BlockSpec((1, RC * K, DOUT), lambda b, i: (b, i, 0)),
    ],
    out_specs=pl.BlockSpec((1, RC, DOUT), lambda b, i: (b, i, 0)),
    out_shape=jax.ShapeDtypeStruct((BG, N, DOUT), jnp.float32),
)


def kernel(x, pos, W_qkv, W1, b1, W2, b2, A1, a1, A2, a2):
    posp = jnp.pad(pos, ((0, 0), (0, 0), (0, 8 - 3)))
    post = jnp.swapaxes(posp, 1, 2)
    w1p = jnp.pad(W1, ((0, 8 - 3), (0, 0)))
    gather = _make_gather(BG * N * K)
    tabs = []
    for g in range(NB):
        s = slice(g * BG, (g + 1) * BG)
        tabs.append(_prep_call(x[s], posp[s], post[s], W_qkv, w1p,
                               b1.reshape(1, -1), W2, b2.reshape(1, -1),
                               A1, a1.reshape(1, -1), A2, a2.reshape(1, -1)))
    outs = []
    for tab, idx in tabs:
        rows = gather(tab.reshape(BG * N, DOUT), idx.reshape(-1))
        outs.append(_agg_call(rows.reshape(BG, N * K, DOUT)))
    return jnp.concatenate(outs, axis=0)
